# baseline (speedup 1.0000x reference)
import functools

import jax
import jax.numpy as jnp
from jax import lax
from jax.experimental import pallas as pl
from jax.experimental.pallas import tpu as pltpu
from jax.experimental.pallas import tpu_sc as plsc

_N_VOCAB = 1000000
_N_EMBD = 32
_CHUNK = 2048

_FULL_CHUNKS = _N_VOCAB // 128
_TAIL = _N_VOCAB - _FULL_CHUNKS * 128


@functools.cache
def _make_detranspose():
    info = plsc.get_sparse_core_info()
    nw = info.num_cores * info.num_subcores
    n_jobs = -(-_FULL_CHUNKS // nw) * nw
    mesh = plsc.VectorSubcoreMesh(core_axis_name="c", subcore_axis_name="s")

    @functools.partial(
        pl.kernel,
        mesh=mesh,
        out_type=jax.ShapeDtypeStruct((_N_VOCAB // 4, 128), jnp.float32),
        scratch_types=[
            pltpu.VMEM((_N_EMBD, 128), jnp.float32),
            pltpu.VMEM((_N_EMBD, 128), jnp.float32),
            pltpu.VMEM((_N_EMBD, _TAIL), jnp.float32),
            pltpu.VMEM((_TAIL * _N_EMBD // 128, 128), jnp.float32),
        ],
        compiler_params=pltpu.CompilerParams(needs_layout_passes=False),
    )
    def k1(tt, out, tin, rout, tin_t, rout_t):
        wid = lax.axis_index("s") * info.num_cores + lax.axis_index("c")
        iot = lax.iota(jnp.int32, 16)

        def scatter_rows(dst, src, ncols):
            for e in range(_N_EMBD):
                for k in range(ncols // 16):
                    v = src[e, pl.ds(16 * k, 16)]
                    flat = (iot + 16 * k) * _N_EMBD + e
                    plsc.store_scatter(
                        dst,
                        [lax.shift_right_logical(flat, 7),
                         lax.bitwise_and(flat, 127)],
                        v,
                    )

        def body(j, carry):
            c = wid + j * nw

            @pl.when(c < _FULL_CHUNKS)
            def _():
                pltpu.sync_copy(tt.at[:, pl.ds(c * 128, 128)], tin)
                scatter_rows(rout, tin, 128)
                pltpu.sync_copy(rout, out.at[pl.ds(c * 32, 32)])

            return carry

        lax.fori_loop(0, n_jobs // nw, body, 0)

        @pl.when(wid == nw - 1)
        def _tail():
            base = _FULL_CHUNKS * 128
            pltpu.sync_copy(tt.at[:, pl.ds(base, _TAIL)], tin_t)
            scatter_rows(rout_t, tin_t, _TAIL)
            pltpu.sync_copy(
                rout_t, out.at[pl.ds(base // 4, _TAIL * _N_EMBD // 128)])

    return k1


@functools.cache
def _make_gather(B, D):
    info = plsc.get_sparse_core_info()
    nw = info.num_cores * info.num_subcores
    b_per_w = B // nw
    n_chunks = b_per_w // _CHUNK
    assert b_per_w % _CHUNK == 0
    mesh = plsc.VectorSubcoreMesh(core_axis_name="c", subcore_axis_name="s")

    @functools.partial(
        pl.kernel,
        mesh=mesh,
        out_type=jax.ShapeDtypeStruct((B, D), jnp.float32),
        scratch_types=[
            pltpu.VMEM((_CHUNK,), jnp.int32),
            pltpu.VMEM((_CHUNK, D), jnp.float32),
            pltpu.SemaphoreType.DMA,
        ],
        compiler_params=pltpu.CompilerParams(use_tc_tiling_on_sc=False),
    )
    def k2(idx_hbm, table_hbm, out_hbm, idx_v, rows_v, sem):
        wid = lax.axis_index("s") * info.num_cores + lax.axis_index("c")
        base = wid * b_per_w

        def body(j, carry):
            off = base + j * _CHUNK
            pltpu.sync_copy(idx_hbm.at[pl.ds(off, _CHUNK)], idx_v)
            pltpu.async_copy(table_hbm.at[idx_v], rows_v, sem).wait()
            pltpu.sync_copy(rows_v, out_hbm.at[pl.ds(off, _CHUNK)])
            return carry

        lax.fori_loop(0, n_chunks, body, 0)

    return k2


def kernel(inputs, wte):
    s0, s1 = inputs.shape
    idx = inputs.reshape(s0 * s1).astype(jnp.int32)
    table_lin = _make_detranspose()(wte.T).reshape(_N_VOCAB, _N_EMBD)
    gathered = _make_gather(s0 * s1, _N_EMBD)(idx, table_lin)
    return (gathered.reshape(s0, s1, _N_EMBD), wte)


# speedup vs baseline: 1.2676x; 1.2676x over previous
import functools

import jax
import jax.numpy as jnp
from jax import lax
from jax.experimental import pallas as pl
from jax.experimental.pallas import tpu as pltpu
from jax.experimental.pallas import tpu_sc as plsc

_N_VOCAB = 1000000
_N_EMBD = 32
_CHUNK = 2048

_W = 512
_FULL_CHUNKS = _N_VOCAB // _W
_TAIL = _N_VOCAB - _FULL_CHUNKS * _W


@functools.cache
def _make_detranspose():
    info = plsc.get_sparse_core_info()
    nw = info.num_cores * info.num_subcores
    max_j2 = (-(-_FULL_CHUNKS // nw) + 1) // 2
    mesh = plsc.VectorSubcoreMesh(core_axis_name="c", subcore_axis_name="s")

    @functools.partial(
        pl.kernel,
        mesh=mesh,
        out_type=jax.ShapeDtypeStruct((_N_VOCAB * _N_EMBD,), jnp.float32),
        scratch_types=[
            pltpu.VMEM((_N_EMBD, _W), jnp.float32),
            pltpu.VMEM((_N_EMBD, _W), jnp.float32),
            pltpu.VMEM((_W * _N_EMBD,), jnp.float32),
            pltpu.VMEM((_W * _N_EMBD,), jnp.float32),
            pltpu.VMEM((_N_EMBD, _TAIL), jnp.float32),
            pltpu.VMEM((_TAIL * _N_EMBD,), jnp.float32),
            pltpu.SemaphoreType.DMA,
            pltpu.SemaphoreType.DMA,
            pltpu.SemaphoreType.DMA,
            pltpu.SemaphoreType.DMA,
        ],
        compiler_params=pltpu.CompilerParams(needs_layout_passes=False),
    )
    def k1(tt, out, tin0, tin1, rout0, rout1, tin_t, rout_t,
           isem0, isem1, osem0, osem1):
        wid = lax.axis_index("s") * info.num_cores + lax.axis_index("c")
        tin = (tin0, tin1)
        rout = (rout0, rout1)
        isem = (isem0, isem1)
        osem = (osem0, osem1)
        iot32 = lax.iota(jnp.int32, 16) * _N_EMBD
        n_valid = (_FULL_CHUNKS - wid + nw - 1) // nw

        def in_slice(c):
            return tt.at[:, pl.ds(c * _W, _W)]

        def out_slice(c):
            return out.at[pl.ds(c * (_W * _N_EMBD), _W * _N_EMBD)]

        def start_in(j, b):
            c = wid + j * nw

            @pl.when(c < _FULL_CHUNKS)
            def _():
                pltpu.async_copy(in_slice(c), tin[b], isem[b])

        def transpose_chunk(src, dst, ncols):
            def kbody(k, carry):
                base = iot32 + k * 16 * _N_EMBD

                def ebody(e, carry2):
                    v = src[e, pl.ds(k * 16, 16)]
                    plsc.store_scatter(dst, [base + e], v)
                    return carry2

                lax.fori_loop(0, _N_EMBD, ebody, 0, unroll=8)
                return carry

            lax.fori_loop(0, ncols // 16, kbody, 0)

        start_in(0, 0)
        start_in(1, 1)

        def body2(j2, carry):
            for b in range(2):
                j = j2 * 2 + b
                c = wid + j * nw

                @pl.when(c < _FULL_CHUNKS)
                def _():
                    @pl.when(j2 >= 1)
                    def _():
                        pltpu.make_async_copy(
                            rout[b], out_slice(c), osem[b]).wait()

                    pltpu.make_async_copy(in_slice(c), tin[b], isem[b]).wait()
                    transpose_chunk(tin[b], rout[b], _W)
                    pltpu.async_copy(rout[b], out_slice(c), osem[b])
                    start_in(j + 2, b)

            return carry

        lax.fori_loop(0, max_j2, body2, 0)

        @pl.when(n_valid >= 1)
        def _drain0():
            pltpu.make_async_copy(rout[0], out_slice(0), osem[0]).wait()

        @pl.when(n_valid >= 2)
        def _drain1():
            pltpu.make_async_copy(rout[1], out_slice(0), osem[1]).wait()

        @pl.when(wid == nw - 1)
        def _tail():
            base = _FULL_CHUNKS * _W
            pltpu.sync_copy(tt.at[:, pl.ds(base, _TAIL)], tin_t)
            transpose_chunk(tin_t, rout_t, _TAIL)
            pltpu.sync_copy(
                rout_t,
                out.at[pl.ds(base * _N_EMBD, _TAIL * _N_EMBD)])

    return k1


@functools.cache
def _make_gather(B, D):
    info = plsc.get_sparse_core_info()
    nw = info.num_cores * info.num_subcores
    b_per_w = B // nw
    n_chunks = b_per_w // _CHUNK
    assert b_per_w % _CHUNK == 0
    mesh = plsc.VectorSubcoreMesh(core_axis_name="c", subcore_axis_name="s")

    @functools.partial(
        pl.kernel,
        mesh=mesh,
        out_type=jax.ShapeDtypeStruct((B, D), jnp.float32),
        scratch_types=[
            pltpu.VMEM((_CHUNK,), jnp.int32),
            pltpu.VMEM((_CHUNK, D), jnp.float32),
            pltpu.SemaphoreType.DMA,
        ],
        compiler_params=pltpu.CompilerParams(use_tc_tiling_on_sc=False),
    )
    def k2(idx_hbm, table_hbm, out_hbm, idx_v, rows_v, sem):
        wid = lax.axis_index("s") * info.num_cores + lax.axis_index("c")
        base = wid * b_per_w

        def body(j, carry):
            off = base + j * _CHUNK
            pltpu.sync_copy(idx_hbm.at[pl.ds(off, _CHUNK)], idx_v)
            pltpu.async_copy(table_hbm.at[idx_v], rows_v, sem).wait()
            pltpu.sync_copy(rows_v, out_hbm.at[pl.ds(off, _CHUNK)])
            return carry

        lax.fori_loop(0, n_chunks, body, 0)

    return k2


def kernel(inputs, wte):
    s0, s1 = inputs.shape
    idx = inputs.reshape(s0 * s1).astype(jnp.int32)
    table_lin = _make_detranspose()(wte.T).reshape(_N_VOCAB, _N_EMBD)
    gathered = _make_gather(s0 * s1, _N_EMBD)(idx, table_lin)
    return (gathered.reshape(s0, s1, _N_EMBD), wte)
